# baseline (speedup 1.0000x reference)
import functools

import jax
import jax.numpy as jnp
from jax import lax
from jax.experimental import pallas as pl
from jax.experimental.pallas import tpu as pltpu
from jax.experimental.pallas import tpu_sc as plsc

N = 10000
E = 320000
D = 128
NT = 16
CHUNK = 64
EPT = 20480
E_PAD = EPT * NT
NP = 10016
ROWS_PT = 632
LAST_R0 = 15 * ROWS_PT
LAST_ROWS = NP - LAST_R0
NCHUNK = EPT // CHUNK

_MESH = plsc.VectorSubcoreMesh(core_axis_name="c", subcore_axis_name="s")


def _copy_idx(src_ref, dst_ref):
    def body(i, _):
        sl = pl.ds(i * 16, 16)
        dst_ref[sl] = src_ref[sl]
        return 0
    lax.fori_loop(0, CHUNK // 16, body, 0, unroll=True)


def _s1_body(swap, xsrc, dste, srce, wre, wie, z128, out,
             acc,
             idx0, idx1, src0, src1, wr0, wr1, wi0, wi1, wa0, wa1,
             gr0, gr1, ms0, ms1, sx0, sx1,
             sl0, sl1, sg0, sg1, ss0, ss1):
    c = lax.axis_index("c")
    s = lax.axis_index("s")
    r0 = s * ROWS_PT

    @pl.when(s < NT - 1)
    def _():
        pltpu.sync_copy(z128.at[pl.ds(r0, ROWS_PT)], acc.at[pl.ds(r0, ROWS_PT)])

    @pl.when(s == NT - 1)
    def _():
        pltpu.sync_copy(z128.at[pl.ds(LAST_R0, LAST_ROWS)],
                        acc.at[pl.ds(LAST_R0, LAST_ROWS)])

    plsc.subcore_barrier()

    wr_core = 1 if swap else 0
    base0 = s * EPT
    idxv = [idx0, idx1]
    srcv = [src0, src1]
    wrv = [wr0, wr1]
    wiv = [wi0, wi1]
    wA = [wa0, wa1]
    grow = [gr0, gr1]
    msg = [ms0, ms1]
    scx = [sx0, sx1]
    sem_ld = [sl0, sl1]
    sem_g = [sg0, sg1]
    sem_s = [ss0, ss1]

    def loads(k, b):
        e0 = base0 + k * CHUNK
        pltpu.async_copy(dste.at[pl.ds(e0, CHUNK)], idxv[b], sem_ld[b])
        pltpu.async_copy(srce.at[pl.ds(e0, CHUNK)], srcv[b], sem_ld[b])
        pltpu.async_copy(wre.at[pl.ds(e0, CHUNK)], wrv[b], sem_ld[b])
        pltpu.async_copy(wie.at[pl.ds(e0, CHUNK)], wiv[b], sem_ld[b])

    def wait_loads(b):
        pltpu.make_async_copy(dste.at[pl.ds(0, CHUNK)], idxv[b], sem_ld[b]).wait()
        pltpu.make_async_copy(srce.at[pl.ds(0, CHUNK)], srcv[b], sem_ld[b]).wait()
        pltpu.make_async_copy(wre.at[pl.ds(0, CHUNK)], wrv[b], sem_ld[b]).wait()
        pltpu.make_async_copy(wie.at[pl.ds(0, CHUNK)], wiv[b], sem_ld[b]).wait()

    def issue_gather(b):
        pltpu.async_copy(xsrc.at[idxv[b]], grow[b], sem_g[b])

    def wait_gather(b):
        pltpu.make_async_copy(xsrc.at[idxv[b]], grow[b], sem_g[b]).wait()

    def compute(b):
        def selw(g, _):
            sl16 = pl.ds(g * 16, 16)
            wA[b][sl16] = jnp.where(c == wr_core, wrv[b][sl16], wiv[b][sl16])
            return 0
        lax.fori_loop(0, CHUNK // 16, selw, 0)

        @plsc.parallel_loop(0, CHUNK, unroll=6)
        def _edge(e):
            a = wA[b][pl.ds(e, 16)][0]
            for j in range(D // 16):
                sl = pl.ds(j * 16, 16)
                msg[b][e, sl] = a * grow[b][e, sl]

    def issue_scatter(b):
        _copy_idx(srcv[b], scx[b])
        pltpu.async_copy(msg[b], acc.at[scx[b]], sem_s[b], add=True)

    def wait_scatter(b):
        pltpu.make_async_copy(msg[b], acc.at[scx[b]], sem_s[b]).wait()

    loads(0, 0)
    loads(1, 1)
    wait_loads(0)
    issue_gather(0)

    @pl.loop(0, NCHUNK, step=2)
    def _pipeline(k):
        for b in (0, 1):
            o = 1 - b
            kk = k + b

            @pl.when(kk + 1 < NCHUNK)
            def _():
                wait_loads(o)
                issue_gather(o)

            @pl.when(kk >= 2)
            def _():
                wait_scatter(b)

            wait_gather(b)
            compute(b)
            issue_scatter(b)

            @pl.when(kk + 2 < NCHUNK)
            def _():
                loads(kk + 2, b)

    wait_scatter(0)
    wait_scatter(1)
    plsc.subcore_barrier()

    @pl.when(s < NT - 1)
    def _():
        pltpu.sync_copy(acc.at[pl.ds(r0, ROWS_PT)],
                        out.at[pl.ds(c * NP + r0, ROWS_PT)])

    @pl.when(s == NT - 1)
    def _():
        pltpu.sync_copy(acc.at[pl.ds(LAST_R0, LAST_ROWS)],
                        out.at[pl.ds(c * NP + LAST_R0, LAST_ROWS)])


def _s2_body(s1a, s1b, dste, srce, wre, wie, z128, out,
             acc,
             idx0, idx1, src0, src1, wr0, wr1, wi0, wi1,
             wa0, wa1, wb0, wb1,
             ga0, ga1, gb0, gb1, ms0, ms1, sx0, sx1,
             sl0, sl1, sg0, sg1, ss0, ss1):
    c = lax.axis_index("c")
    s = lax.axis_index("s")
    r0 = s * ROWS_PT

    @pl.when(s < NT - 1)
    def _():
        pltpu.sync_copy(z128.at[pl.ds(r0, ROWS_PT)], acc.at[pl.ds(r0, ROWS_PT)])

    @pl.when(s == NT - 1)
    def _():
        pltpu.sync_copy(z128.at[pl.ds(LAST_R0, LAST_ROWS)],
                        acc.at[pl.ds(LAST_R0, LAST_ROWS)])

    plsc.subcore_barrier()

    base0 = s * EPT
    goff = c * NP
    idxv = [idx0, idx1]
    srcv = [src0, src1]
    wrv = [wr0, wr1]
    wiv = [wi0, wi1]
    wA = [wa0, wa1]
    wB = [wb0, wb1]
    gA = [ga0, ga1]
    gB = [gb0, gb1]
    msg = [ms0, ms1]
    scx = [sx0, sx1]
    sem_ld = [sl0, sl1]
    sem_g = [sg0, sg1]
    sem_s = [ss0, ss1]

    def loads(k, b):
        e0 = base0 + k * CHUNK
        pltpu.async_copy(dste.at[pl.ds(e0, CHUNK)], idxv[b], sem_ld[b])
        pltpu.async_copy(srce.at[pl.ds(e0, CHUNK)], srcv[b], sem_ld[b])
        pltpu.async_copy(wre.at[pl.ds(e0, CHUNK)], wrv[b], sem_ld[b])
        pltpu.async_copy(wie.at[pl.ds(e0, CHUNK)], wiv[b], sem_ld[b])

    def wait_loads(b):
        pltpu.make_async_copy(dste.at[pl.ds(0, CHUNK)], idxv[b], sem_ld[b]).wait()
        pltpu.make_async_copy(srce.at[pl.ds(0, CHUNK)], srcv[b], sem_ld[b]).wait()
        pltpu.make_async_copy(wre.at[pl.ds(0, CHUNK)], wrv[b], sem_ld[b]).wait()
        pltpu.make_async_copy(wie.at[pl.ds(0, CHUNK)], wiv[b], sem_ld[b]).wait()

    def issue_gathers(b):
        def adjA(i, _):
            sl = pl.ds(i * 16, 16)
            idxv[b][sl] = idxv[b][sl] + goff
            return 0
        lax.fori_loop(0, CHUNK // 16, adjA, 0, unroll=True)
        pltpu.async_copy(s1a.at[idxv[b]], gA[b], sem_g[b])
        pltpu.async_copy(s1b.at[idxv[b]], gB[b], sem_g[b])

    def wait_gathers(b):
        pltpu.make_async_copy(s1a.at[idxv[b]], gA[b], sem_g[b]).wait()
        pltpu.make_async_copy(s1b.at[idxv[b]], gB[b], sem_g[b]).wait()

    def compute(b):
        def selw(g, _):
            sl16 = pl.ds(g * 16, 16)
            wr16 = wrv[b][sl16]
            wi16 = wiv[b][sl16]
            wA[b][sl16] = jnp.where(c == 0, wr16, wi16)
            wB[b][sl16] = jnp.where(c == 0, -wi16, wr16)
            return 0
        lax.fori_loop(0, CHUNK // 16, selw, 0)

        @plsc.parallel_loop(0, CHUNK, unroll=6)
        def _edge(e):
            a = wA[b][pl.ds(e, 16)][0]
            bb = wB[b][pl.ds(e, 16)][0]
            for j in range(D // 16):
                sl = pl.ds(j * 16, 16)
                msg[b][e, sl] = a * gA[b][e, sl] + bb * gB[b][e, sl]

    def issue_scatter(b):
        _copy_idx(srcv[b], scx[b])
        pltpu.async_copy(msg[b], acc.at[scx[b]], sem_s[b], add=True)

    def wait_scatter(b):
        pltpu.make_async_copy(msg[b], acc.at[scx[b]], sem_s[b]).wait()

    loads(0, 0)
    loads(1, 1)
    wait_loads(0)
    issue_gathers(0)

    @pl.loop(0, NCHUNK, step=2)
    def _pipeline(k):
        for b in (0, 1):
            o = 1 - b
            kk = k + b

            @pl.when(kk + 1 < NCHUNK)
            def _():
                wait_loads(o)
                issue_gathers(o)

            @pl.when(kk >= 2)
            def _():
                wait_scatter(b)

            wait_gathers(b)
            compute(b)
            issue_scatter(b)

            @pl.when(kk + 2 < NCHUNK)
            def _():
                loads(kk + 2, b)

    wait_scatter(0)
    wait_scatter(1)
    plsc.subcore_barrier()

    @pl.when(s < NT - 1)
    def _():
        pltpu.sync_copy(acc.at[pl.ds(r0, ROWS_PT)],
                        out.at[pl.ds(c * NP + r0, ROWS_PT)])

    @pl.when(s == NT - 1)
    def _():
        pltpu.sync_copy(acc.at[pl.ds(LAST_R0, LAST_ROWS)],
                        out.at[pl.ds(c * NP + LAST_R0, LAST_ROWS)])


def _tc_body(xr, xi, s1a, s1b, u2, W, bias, outr, outi):
    W02 = W[0] - W[2]
    W1 = W[1]
    W22 = 2.0 * W[2]
    b = bias[0, :]

    rr = s1a[0]
    ri = s1a[1]
    ii = s1b[0]
    ir = s1b[1]

    dot = functools.partial(jnp.dot, preferred_element_type=jnp.float32)
    outr[...] = dot(xr[...], W02) + dot(ii - rr, W1) + dot(u2[0], W22) + b
    outi[...] = dot(xi[...], W02) - dot(ir + ri, W1) + dot(u2[1], W22) + b


def _sc_scratch_base():
    return [
        pltpu.VMEM_SHARED((NP, D), jnp.float32),
        pltpu.VMEM((CHUNK,), jnp.int32), pltpu.VMEM((CHUNK,), jnp.int32),
        pltpu.VMEM((CHUNK,), jnp.int32), pltpu.VMEM((CHUNK,), jnp.int32),
        pltpu.VMEM((CHUNK,), jnp.float32), pltpu.VMEM((CHUNK,), jnp.float32),
        pltpu.VMEM((CHUNK,), jnp.float32), pltpu.VMEM((CHUNK,), jnp.float32),
    ]


def _s1_scratch():
    return _sc_scratch_base() + [
        pltpu.VMEM((CHUNK + 16,), jnp.float32),
        pltpu.VMEM((CHUNK + 16,), jnp.float32),
        pltpu.VMEM((CHUNK, D), jnp.float32), pltpu.VMEM((CHUNK, D), jnp.float32),
        pltpu.VMEM((CHUNK, D), jnp.float32), pltpu.VMEM((CHUNK, D), jnp.float32),
        pltpu.VMEM((CHUNK,), jnp.int32), pltpu.VMEM((CHUNK,), jnp.int32),
    ] + [pltpu.SemaphoreType.DMA for _ in range(6)]


def _s2_scratch():
    return _sc_scratch_base() + [
        pltpu.VMEM((CHUNK + 16,), jnp.float32),
        pltpu.VMEM((CHUNK + 16,), jnp.float32),
        pltpu.VMEM((CHUNK + 16,), jnp.float32),
        pltpu.VMEM((CHUNK + 16,), jnp.float32),
        pltpu.VMEM((CHUNK, D), jnp.float32), pltpu.VMEM((CHUNK, D), jnp.float32),
        pltpu.VMEM((CHUNK, D), jnp.float32), pltpu.VMEM((CHUNK, D), jnp.float32),
        pltpu.VMEM((CHUNK, D), jnp.float32), pltpu.VMEM((CHUNK, D), jnp.float32),
        pltpu.VMEM((CHUNK,), jnp.int32), pltpu.VMEM((CHUNK,), jnp.int32),
    ] + [pltpu.SemaphoreType.DMA for _ in range(6)]


@jax.jit
def kernel(x_real, x_imag, edge_index, norm_real, norm_imag, W, bias):
    src = edge_index[0].astype(jnp.int32)
    dst = edge_index[1].astype(jnp.int32)
    wr = norm_real.astype(jnp.float32)
    wi = norm_imag.astype(jnp.float32)

    npad = E_PAD - E
    spread = (jnp.arange(npad, dtype=jnp.int32) * 61) % N
    dst_p = jnp.concatenate([dst, spread])
    src_p = jnp.concatenate([src, spread])
    zpad = jnp.zeros((npad,), jnp.float32)
    wr_p = jnp.concatenate([wr, zpad])
    wi_p = jnp.concatenate([wi, zpad])

    zrows = jnp.zeros((NP - N, D), jnp.float32)
    xrp = jnp.concatenate([x_real, zrows], axis=0)
    xip = jnp.concatenate([x_imag, zrows], axis=0)
    z128 = jnp.zeros((NP, D), jnp.float32)

    s1a = pl.kernel(
        functools.partial(_s1_body, False),
        out_type=jax.ShapeDtypeStruct((2 * NP, D), jnp.float32),
        mesh=_MESH, scratch_types=_s1_scratch(),
    )(xrp, dst_p, src_p, wr_p, wi_p, z128)
    s1b = pl.kernel(
        functools.partial(_s1_body, True),
        out_type=jax.ShapeDtypeStruct((2 * NP, D), jnp.float32),
        mesh=_MESH, scratch_types=_s1_scratch(),
    )(xip, dst_p, src_p, wr_p, wi_p, z128)

    u2 = pl.kernel(
        _s2_body,
        out_type=jax.ShapeDtypeStruct((2 * NP, D), jnp.float32),
        mesh=_MESH, scratch_types=_s2_scratch(),
    )(s1a, s1b, dst_p, src_p, wr_p, wi_p, z128)

    RB = 1000
    grid = N // RB
    out_real, out_imag = pl.pallas_call(
        _tc_body,
        grid=(grid,),
        in_specs=[
            pl.BlockSpec((RB, D), lambda i: (i, 0)),
            pl.BlockSpec((RB, D), lambda i: (i, 0)),
            pl.BlockSpec((2, RB, D), lambda i: (0, i, 0)),
            pl.BlockSpec((2, RB, D), lambda i: (0, i, 0)),
            pl.BlockSpec((2, RB, D), lambda i: (0, i, 0)),
            pl.BlockSpec((3, D, D), lambda i: (0, 0, 0)),
            pl.BlockSpec((1, D), lambda i: (0, 0)),
        ],
        out_specs=[
            pl.BlockSpec((RB, D), lambda i: (i, 0)),
            pl.BlockSpec((RB, D), lambda i: (i, 0)),
        ],
        out_shape=[
            jax.ShapeDtypeStruct((N, D), jnp.float32),
            jax.ShapeDtypeStruct((N, D), jnp.float32),
        ],
    )(x_real, x_imag,
      s1a.reshape(2, NP, D), s1b.reshape(2, NP, D),
      u2.reshape(2, NP, D), W, bias.reshape(1, D))

    return out_real, out_imag


# speedup vs baseline: 1.0081x; 1.0081x over previous
import functools

import jax
import jax.numpy as jnp
from jax import lax
from jax.experimental import pallas as pl
from jax.experimental.pallas import tpu as pltpu
from jax.experimental.pallas import tpu_sc as plsc

N = 10000
E = 320000
D = 128
NT = 16
CHUNK = 64
EPT = 20480
E_PAD = EPT * NT
NP = 10016
ROWS_PT = 632
LAST_R0 = 15 * ROWS_PT
LAST_ROWS = NP - LAST_R0
NCHUNK = EPT // CHUNK

_MESH = plsc.VectorSubcoreMesh(core_axis_name="c", subcore_axis_name="s")


def _copy_idx(src_ref, dst_ref):
    def body(i, _):
        sl = pl.ds(i * 16, 16)
        dst_ref[sl] = src_ref[sl]
        return 0
    lax.fori_loop(0, CHUNK // 16, body, 0, unroll=True)


def _s1_body(swap, xsrc, dste, srce, wre, wie, z128, out,
             acc,
             idx0, idx1, src0, src1, wr0, wr1, wi0, wi1, wa0, wa1,
             gr0, gr1, ms0, ms1, sx0, sx1,
             sl0, sl1, sg0, sg1, ss0, ss1):
    c = lax.axis_index("c")
    s = lax.axis_index("s")
    r0 = s * ROWS_PT

    @pl.when(s < NT - 1)
    def _():
        pltpu.sync_copy(z128.at[pl.ds(r0, ROWS_PT)], acc.at[pl.ds(r0, ROWS_PT)])

    @pl.when(s == NT - 1)
    def _():
        pltpu.sync_copy(z128.at[pl.ds(LAST_R0, LAST_ROWS)],
                        acc.at[pl.ds(LAST_R0, LAST_ROWS)])

    plsc.subcore_barrier()

    wr_core = 1 if swap else 0
    base0 = s * EPT
    idxv = [idx0, idx1]
    srcv = [src0, src1]
    wrv = [wr0, wr1]
    wiv = [wi0, wi1]
    wA = [wa0, wa1]
    grow = [gr0, gr1]
    msg = [ms0, ms1]
    scx = [sx0, sx1]
    sem_ld = [sl0, sl1]
    sem_g = [sg0, sg1]
    sem_s = [ss0, ss1]

    def loads(k, b):
        e0 = base0 + k * CHUNK
        pltpu.async_copy(dste.at[pl.ds(e0, CHUNK)], idxv[b], sem_ld[b])
        pltpu.async_copy(srce.at[pl.ds(e0, CHUNK)], srcv[b], sem_ld[b])
        pltpu.async_copy(wre.at[pl.ds(e0, CHUNK)], wrv[b], sem_ld[b])
        pltpu.async_copy(wie.at[pl.ds(e0, CHUNK)], wiv[b], sem_ld[b])

    def wait_loads(b):
        pltpu.make_async_copy(dste.at[pl.ds(0, CHUNK)], idxv[b], sem_ld[b]).wait()
        pltpu.make_async_copy(srce.at[pl.ds(0, CHUNK)], srcv[b], sem_ld[b]).wait()
        pltpu.make_async_copy(wre.at[pl.ds(0, CHUNK)], wrv[b], sem_ld[b]).wait()
        pltpu.make_async_copy(wie.at[pl.ds(0, CHUNK)], wiv[b], sem_ld[b]).wait()

    def issue_gather(b):
        pltpu.async_copy(xsrc.at[idxv[b]], grow[b], sem_g[b])

    def wait_gather(b):
        pltpu.make_async_copy(xsrc.at[idxv[b]], grow[b], sem_g[b]).wait()

    def compute(b):
        def selw(g, _):
            sl16 = pl.ds(g * 16, 16)
            wA[b][sl16] = jnp.where(c == wr_core, wrv[b][sl16], wiv[b][sl16])
            return 0
        lax.fori_loop(0, CHUNK // 16, selw, 0)

        @plsc.parallel_loop(0, CHUNK, unroll=4)
        def _edge(e):
            a = wA[b][pl.ds(e, 16)][0]
            for j in range(D // 16):
                sl = pl.ds(j * 16, 16)
                msg[b][e, sl] = a * grow[b][e, sl]

    def issue_scatter(b):
        _copy_idx(srcv[b], scx[b])
        pltpu.async_copy(msg[b], acc.at[scx[b]], sem_s[b], add=True)

    def wait_scatter(b):
        pltpu.make_async_copy(msg[b], acc.at[scx[b]], sem_s[b]).wait()

    loads(0, 0)
    loads(1, 1)
    wait_loads(0)
    issue_gather(0)

    @pl.loop(0, NCHUNK, step=2)
    def _pipeline(k):
        for b in (0, 1):
            o = 1 - b
            kk = k + b

            @pl.when(kk + 1 < NCHUNK)
            def _():
                wait_loads(o)
                issue_gather(o)

            @pl.when(kk >= 2)
            def _():
                wait_scatter(b)

            wait_gather(b)
            compute(b)
            issue_scatter(b)

            @pl.when(kk + 2 < NCHUNK)
            def _():
                loads(kk + 2, b)

    wait_scatter(0)
    wait_scatter(1)
    plsc.subcore_barrier()

    @pl.when(s < NT - 1)
    def _():
        pltpu.sync_copy(acc.at[pl.ds(r0, ROWS_PT)],
                        out.at[pl.ds(c * NP + r0, ROWS_PT)])

    @pl.when(s == NT - 1)
    def _():
        pltpu.sync_copy(acc.at[pl.ds(LAST_R0, LAST_ROWS)],
                        out.at[pl.ds(c * NP + LAST_R0, LAST_ROWS)])


def _s2_body(s1a, s1b, dste, srce, wre, wie, z128, out,
             acc,
             idx0, idx1, src0, src1, wr0, wr1, wi0, wi1,
             wa0, wa1, wb0, wb1,
             ga0, ga1, gb0, gb1, ms0, ms1, sx0, sx1,
             sl0, sl1, sg0, sg1, ss0, ss1):
    c = lax.axis_index("c")
    s = lax.axis_index("s")
    r0 = s * ROWS_PT

    @pl.when(s < NT - 1)
    def _():
        pltpu.sync_copy(z128.at[pl.ds(r0, ROWS_PT)], acc.at[pl.ds(r0, ROWS_PT)])

    @pl.when(s == NT - 1)
    def _():
        pltpu.sync_copy(z128.at[pl.ds(LAST_R0, LAST_ROWS)],
                        acc.at[pl.ds(LAST_R0, LAST_ROWS)])

    plsc.subcore_barrier()

    base0 = s * EPT
    goff = c * NP
    idxv = [idx0, idx1]
    srcv = [src0, src1]
    wrv = [wr0, wr1]
    wiv = [wi0, wi1]
    wA = [wa0, wa1]
    wB = [wb0, wb1]
    gA = [ga0, ga1]
    gB = [gb0, gb1]
    msg = [ms0, ms1]
    scx = [sx0, sx1]
    sem_ld = [sl0, sl1]
    sem_g = [sg0, sg1]
    sem_s = [ss0, ss1]

    def loads(k, b):
        e0 = base0 + k * CHUNK
        pltpu.async_copy(dste.at[pl.ds(e0, CHUNK)], idxv[b], sem_ld[b])
        pltpu.async_copy(srce.at[pl.ds(e0, CHUNK)], srcv[b], sem_ld[b])
        pltpu.async_copy(wre.at[pl.ds(e0, CHUNK)], wrv[b], sem_ld[b])
        pltpu.async_copy(wie.at[pl.ds(e0, CHUNK)], wiv[b], sem_ld[b])

    def wait_loads(b):
        pltpu.make_async_copy(dste.at[pl.ds(0, CHUNK)], idxv[b], sem_ld[b]).wait()
        pltpu.make_async_copy(srce.at[pl.ds(0, CHUNK)], srcv[b], sem_ld[b]).wait()
        pltpu.make_async_copy(wre.at[pl.ds(0, CHUNK)], wrv[b], sem_ld[b]).wait()
        pltpu.make_async_copy(wie.at[pl.ds(0, CHUNK)], wiv[b], sem_ld[b]).wait()

    def issue_gathers(b):
        def adjA(i, _):
            sl = pl.ds(i * 16, 16)
            idxv[b][sl] = idxv[b][sl] + goff
            return 0
        lax.fori_loop(0, CHUNK // 16, adjA, 0, unroll=True)
        pltpu.async_copy(s1a.at[idxv[b]], gA[b], sem_g[b])
        pltpu.async_copy(s1b.at[idxv[b]], gB[b], sem_g[b])

    def wait_gathers(b):
        pltpu.make_async_copy(s1a.at[idxv[b]], gA[b], sem_g[b]).wait()
        pltpu.make_async_copy(s1b.at[idxv[b]], gB[b], sem_g[b]).wait()

    def compute(b):
        def selw(g, _):
            sl16 = pl.ds(g * 16, 16)
            wr16 = wrv[b][sl16]
            wi16 = wiv[b][sl16]
            wA[b][sl16] = jnp.where(c == 0, wr16, wi16)
            wB[b][sl16] = jnp.where(c == 0, -wi16, wr16)
            return 0
        lax.fori_loop(0, CHUNK // 16, selw, 0)

        @plsc.parallel_loop(0, CHUNK, unroll=4)
        def _edge(e):
            a = wA[b][pl.ds(e, 16)][0]
            bb = wB[b][pl.ds(e, 16)][0]
            for j in range(D // 16):
                sl = pl.ds(j * 16, 16)
                msg[b][e, sl] = a * gA[b][e, sl] + bb * gB[b][e, sl]

    def issue_scatter(b):
        _copy_idx(srcv[b], scx[b])
        pltpu.async_copy(msg[b], acc.at[scx[b]], sem_s[b], add=True)

    def wait_scatter(b):
        pltpu.make_async_copy(msg[b], acc.at[scx[b]], sem_s[b]).wait()

    loads(0, 0)
    loads(1, 1)
    wait_loads(0)
    issue_gathers(0)

    @pl.loop(0, NCHUNK, step=2)
    def _pipeline(k):
        for b in (0, 1):
            o = 1 - b
            kk = k + b

            @pl.when(kk + 1 < NCHUNK)
            def _():
                wait_loads(o)
                issue_gathers(o)

            @pl.when(kk >= 2)
            def _():
                wait_scatter(b)

            wait_gathers(b)
            compute(b)
            issue_scatter(b)

            @pl.when(kk + 2 < NCHUNK)
            def _():
                loads(kk + 2, b)

    wait_scatter(0)
    wait_scatter(1)
    plsc.subcore_barrier()

    @pl.when(s < NT - 1)
    def _():
        pltpu.sync_copy(acc.at[pl.ds(r0, ROWS_PT)],
                        out.at[pl.ds(c * NP + r0, ROWS_PT)])

    @pl.when(s == NT - 1)
    def _():
        pltpu.sync_copy(acc.at[pl.ds(LAST_R0, LAST_ROWS)],
                        out.at[pl.ds(c * NP + LAST_R0, LAST_ROWS)])


def _tc_body(xr, xi, s1a, s1b, u2, W, bias, outr, outi):
    W02 = W[0] - W[2]
    W1 = W[1]
    W22 = 2.0 * W[2]
    b = bias[0, :]

    rr = s1a[0]
    ri = s1a[1]
    ii = s1b[0]
    ir = s1b[1]

    dot = functools.partial(jnp.dot, preferred_element_type=jnp.float32)
    outr[...] = dot(xr[...], W02) + dot(ii - rr, W1) + dot(u2[0], W22) + b
    outi[...] = dot(xi[...], W02) - dot(ir + ri, W1) + dot(u2[1], W22) + b


def _sc_scratch_base():
    return [
        pltpu.VMEM_SHARED((NP, D), jnp.float32),
        pltpu.VMEM((CHUNK,), jnp.int32), pltpu.VMEM((CHUNK,), jnp.int32),
        pltpu.VMEM((CHUNK,), jnp.int32), pltpu.VMEM((CHUNK,), jnp.int32),
        pltpu.VMEM((CHUNK,), jnp.float32), pltpu.VMEM((CHUNK,), jnp.float32),
        pltpu.VMEM((CHUNK,), jnp.float32), pltpu.VMEM((CHUNK,), jnp.float32),
    ]


def _s1_scratch():
    return _sc_scratch_base() + [
        pltpu.VMEM((CHUNK + 16,), jnp.float32),
        pltpu.VMEM((CHUNK + 16,), jnp.float32),
        pltpu.VMEM((CHUNK, D), jnp.float32), pltpu.VMEM((CHUNK, D), jnp.float32),
        pltpu.VMEM((CHUNK, D), jnp.float32), pltpu.VMEM((CHUNK, D), jnp.float32),
        pltpu.VMEM((CHUNK,), jnp.int32), pltpu.VMEM((CHUNK,), jnp.int32),
    ] + [pltpu.SemaphoreType.DMA for _ in range(6)]


def _s2_scratch():
    return _sc_scratch_base() + [
        pltpu.VMEM((CHUNK + 16,), jnp.float32),
        pltpu.VMEM((CHUNK + 16,), jnp.float32),
        pltpu.VMEM((CHUNK + 16,), jnp.float32),
        pltpu.VMEM((CHUNK + 16,), jnp.float32),
        pltpu.VMEM((CHUNK, D), jnp.float32), pltpu.VMEM((CHUNK, D), jnp.float32),
        pltpu.VMEM((CHUNK, D), jnp.float32), pltpu.VMEM((CHUNK, D), jnp.float32),
        pltpu.VMEM((CHUNK, D), jnp.float32), pltpu.VMEM((CHUNK, D), jnp.float32),
        pltpu.VMEM((CHUNK,), jnp.int32), pltpu.VMEM((CHUNK,), jnp.int32),
    ] + [pltpu.SemaphoreType.DMA for _ in range(6)]


@jax.jit
def kernel(x_real, x_imag, edge_index, norm_real, norm_imag, W, bias):
    src = edge_index[0].astype(jnp.int32)
    dst = edge_index[1].astype(jnp.int32)
    wr = norm_real.astype(jnp.float32)
    wi = norm_imag.astype(jnp.float32)

    npad = E_PAD - E
    spread = (jnp.arange(npad, dtype=jnp.int32) * 61) % N
    dst_p = jnp.concatenate([dst, spread])
    src_p = jnp.concatenate([src, spread])
    zpad = jnp.zeros((npad,), jnp.float32)
    wr_p = jnp.concatenate([wr, zpad])
    wi_p = jnp.concatenate([wi, zpad])

    zrows = jnp.zeros((NP - N, D), jnp.float32)
    xrp = jnp.concatenate([x_real, zrows], axis=0)
    xip = jnp.concatenate([x_imag, zrows], axis=0)
    z128 = jnp.zeros((NP, D), jnp.float32)

    s1a = pl.kernel(
        functools.partial(_s1_body, False),
        out_type=jax.ShapeDtypeStruct((2 * NP, D), jnp.float32),
        mesh=_MESH, scratch_types=_s1_scratch(),
    )(xrp, dst_p, src_p, wr_p, wi_p, z128)
    s1b = pl.kernel(
        functools.partial(_s1_body, True),
        out_type=jax.ShapeDtypeStruct((2 * NP, D), jnp.float32),
        mesh=_MESH, scratch_types=_s1_scratch(),
    )(xip, dst_p, src_p, wr_p, wi_p, z128)

    u2 = pl.kernel(
        _s2_body,
        out_type=jax.ShapeDtypeStruct((2 * NP, D), jnp.float32),
        mesh=_MESH, scratch_types=_s2_scratch(),
    )(s1a, s1b, dst_p, src_p, wr_p, wi_p, z128)

    RB = 1000
    grid = N // RB
    out_real, out_imag = pl.pallas_call(
        _tc_body,
        grid=(grid,),
        in_specs=[
            pl.BlockSpec((RB, D), lambda i: (i, 0)),
            pl.BlockSpec((RB, D), lambda i: (i, 0)),
            pl.BlockSpec((2, RB, D), lambda i: (0, i, 0)),
            pl.BlockSpec((2, RB, D), lambda i: (0, i, 0)),
            pl.BlockSpec((2, RB, D), lambda i: (0, i, 0)),
            pl.BlockSpec((3, D, D), lambda i: (0, 0, 0)),
            pl.BlockSpec((1, D), lambda i: (0, 0)),
        ],
        out_specs=[
            pl.BlockSpec((RB, D), lambda i: (i, 0)),
            pl.BlockSpec((RB, D), lambda i: (i, 0)),
        ],
        out_shape=[
            jax.ShapeDtypeStruct((N, D), jnp.float32),
            jax.ShapeDtypeStruct((N, D), jnp.float32),
        ],
    )(x_real, x_imag,
      s1a.reshape(2, NP, D), s1b.reshape(2, NP, D),
      u2.reshape(2, NP, D), W, bias.reshape(1, D))

    return out_real, out_imag


# speedup vs baseline: 1.0132x; 1.0050x over previous
import functools

import jax
import jax.numpy as jnp
from jax import lax
from jax.experimental import pallas as pl
from jax.experimental.pallas import tpu as pltpu
from jax.experimental.pallas import tpu_sc as plsc

N = 10000
E = 320000
D = 128
NT = 16
CHUNK = 64
EPT = 20480
E_PAD = EPT * NT
NP = 10016
ROWS_PT = 632
LAST_R0 = 15 * ROWS_PT
LAST_ROWS = NP - LAST_R0
NCHUNK = EPT // CHUNK

_MESH = plsc.VectorSubcoreMesh(core_axis_name="c", subcore_axis_name="s")


def _copy_idx(src_ref, dst_ref):
    def body(i, _):
        sl = pl.ds(i * 16, 16)
        dst_ref[sl] = src_ref[sl]
        return 0
    lax.fori_loop(0, CHUNK // 16, body, 0, unroll=True)


def _s1_body(swap, xsrc, dste, srce, wre, wie, z128, out,
             acc,
             idx0, idx1, src0, src1, wr0, wr1, wi0, wi1, wa0, wa1,
             gr0, gr1, ms0, ms1, sx0, sx1,
             sl0, sl1, sg0, sg1, ss0, ss1):
    c = lax.axis_index("c")
    s = lax.axis_index("s")
    r0 = s * ROWS_PT

    @pl.when(s < NT - 1)
    def _():
        pltpu.sync_copy(z128.at[pl.ds(r0, ROWS_PT)], acc.at[pl.ds(r0, ROWS_PT)])

    @pl.when(s == NT - 1)
    def _():
        pltpu.sync_copy(z128.at[pl.ds(LAST_R0, LAST_ROWS)],
                        acc.at[pl.ds(LAST_R0, LAST_ROWS)])

    plsc.subcore_barrier()

    wr_core = 1 if swap else 0
    base0 = s * EPT
    idxv = [idx0, idx1]
    srcv = [src0, src1]
    wrv = [wr0, wr1]
    wiv = [wi0, wi1]
    wA = [wa0, wa1]
    grow = [gr0, gr1]
    msg = [ms0, ms1]
    scx = [sx0, sx1]
    sem_ld = [sl0, sl1]
    sem_g = [sg0, sg1]
    sem_s = [ss0, ss1]

    def loads(k, b):
        e0 = base0 + k * CHUNK
        pltpu.async_copy(dste.at[pl.ds(e0, CHUNK)], idxv[b], sem_ld[b])
        pltpu.async_copy(srce.at[pl.ds(e0, CHUNK)], srcv[b], sem_ld[b])
        pltpu.async_copy(wre.at[pl.ds(e0, CHUNK)], wrv[b], sem_ld[b])
        pltpu.async_copy(wie.at[pl.ds(e0, CHUNK)], wiv[b], sem_ld[b])

    def wait_loads(b):
        pltpu.make_async_copy(dste.at[pl.ds(0, CHUNK)], idxv[b], sem_ld[b]).wait()
        pltpu.make_async_copy(srce.at[pl.ds(0, CHUNK)], srcv[b], sem_ld[b]).wait()
        pltpu.make_async_copy(wre.at[pl.ds(0, CHUNK)], wrv[b], sem_ld[b]).wait()
        pltpu.make_async_copy(wie.at[pl.ds(0, CHUNK)], wiv[b], sem_ld[b]).wait()

    def issue_gather(b):
        pltpu.async_copy(xsrc.at[idxv[b]], grow[b], sem_g[b])

    def wait_gather(b):
        pltpu.make_async_copy(xsrc.at[idxv[b]], grow[b], sem_g[b]).wait()

    def compute(b):
        def selw(g, _):
            sl16 = pl.ds(g * 16, 16)
            wA[b][sl16] = jnp.where(c == wr_core, wrv[b][sl16], wiv[b][sl16])
            return 0
        lax.fori_loop(0, CHUNK // 16, selw, 0)

        @plsc.parallel_loop(0, CHUNK, unroll=4)
        def _edge(e):
            a = wA[b][pl.ds(e, 16)][0]
            for j in range(D // 16):
                sl = pl.ds(j * 16, 16)
                msg[b][e, sl] = a * grow[b][e, sl]

    def issue_scatter(b):
        _copy_idx(srcv[b], scx[b])
        pltpu.async_copy(msg[b], acc.at[scx[b]], sem_s[b], add=True)

    def wait_scatter(b):
        pltpu.make_async_copy(msg[b], acc.at[scx[b]], sem_s[b]).wait()

    loads(0, 0)
    loads(1, 1)
    wait_loads(0)
    issue_gather(0)

    @pl.loop(0, NCHUNK, step=2)
    def _pipeline(k):
        for b in (0, 1):
            o = 1 - b
            kk = k + b

            @pl.when(kk + 1 < NCHUNK)
            def _():
                wait_loads(o)
                issue_gather(o)

            @pl.when(kk >= 2)
            def _():
                wait_scatter(b)

            wait_gather(b)
            compute(b)
            issue_scatter(b)

            @pl.when(kk + 2 < NCHUNK)
            def _():
                loads(kk + 2, b)

    wait_scatter(0)
    wait_scatter(1)
    plsc.subcore_barrier()

    @pl.when(s < NT - 1)
    def _():
        pltpu.sync_copy(acc.at[pl.ds(r0, ROWS_PT)],
                        out.at[pl.ds(c * NP + r0, ROWS_PT)])

    @pl.when(s == NT - 1)
    def _():
        pltpu.sync_copy(acc.at[pl.ds(LAST_R0, LAST_ROWS)],
                        out.at[pl.ds(c * NP + LAST_R0, LAST_ROWS)])


def _s2_body(s1a, s1b, dste, srce, wre, wie, z128, out,
             acc,
             idx0, idx1, src0, src1, wr0, wr1, wi0, wi1,
             wa0, wa1, wb0, wb1,
             ga0, ga1, gb0, gb1, ms0, ms1, sx0, sx1,
             sl0, sl1, sg0, sg1, ss0, ss1):
    c = lax.axis_index("c")
    s = lax.axis_index("s")
    r0 = s * ROWS_PT

    @pl.when(s < NT - 1)
    def _():
        pltpu.sync_copy(z128.at[pl.ds(r0, ROWS_PT)], acc.at[pl.ds(r0, ROWS_PT)])

    @pl.when(s == NT - 1)
    def _():
        pltpu.sync_copy(z128.at[pl.ds(LAST_R0, LAST_ROWS)],
                        acc.at[pl.ds(LAST_R0, LAST_ROWS)])

    plsc.subcore_barrier()

    base0 = s * EPT
    goff = c * NP
    idxv = [idx0, idx1]
    srcv = [src0, src1]
    wrv = [wr0, wr1]
    wiv = [wi0, wi1]
    wA = [wa0, wa1]
    wB = [wb0, wb1]
    gA = [ga0, ga1]
    gB = [gb0, gb1]
    msg = [ms0, ms1]
    scx = [sx0, sx1]
    sem_ld = [sl0, sl1]
    sem_g = [sg0, sg1]
    sem_s = [ss0, ss1]

    def loads(k, b):
        e0 = base0 + k * CHUNK
        pltpu.async_copy(dste.at[pl.ds(e0, CHUNK)], idxv[b], sem_ld[b])
        pltpu.async_copy(srce.at[pl.ds(e0, CHUNK)], srcv[b], sem_ld[b])
        pltpu.async_copy(wre.at[pl.ds(e0, CHUNK)], wrv[b], sem_ld[b])
        pltpu.async_copy(wie.at[pl.ds(e0, CHUNK)], wiv[b], sem_ld[b])

    def wait_loads(b):
        pltpu.make_async_copy(dste.at[pl.ds(0, CHUNK)], idxv[b], sem_ld[b]).wait()
        pltpu.make_async_copy(srce.at[pl.ds(0, CHUNK)], srcv[b], sem_ld[b]).wait()
        pltpu.make_async_copy(wre.at[pl.ds(0, CHUNK)], wrv[b], sem_ld[b]).wait()
        pltpu.make_async_copy(wie.at[pl.ds(0, CHUNK)], wiv[b], sem_ld[b]).wait()

    def issue_gathers(b):
        def adjA(i, _):
            sl = pl.ds(i * 16, 16)
            idxv[b][sl] = idxv[b][sl] + goff
            return 0
        lax.fori_loop(0, CHUNK // 16, adjA, 0, unroll=True)
        pltpu.async_copy(s1a.at[idxv[b]], gA[b], sem_g[b])
        pltpu.async_copy(s1b.at[idxv[b]], gB[b], sem_g[b])

    def wait_gathers(b):
        pltpu.make_async_copy(s1a.at[idxv[b]], gA[b], sem_g[b]).wait()
        pltpu.make_async_copy(s1b.at[idxv[b]], gB[b], sem_g[b]).wait()

    def compute(b):
        def selw(g, _):
            sl16 = pl.ds(g * 16, 16)
            wr16 = wrv[b][sl16]
            wi16 = wiv[b][sl16]
            wA[b][sl16] = jnp.where(c == 0, wr16, wi16)
            wB[b][sl16] = jnp.where(c == 0, -wi16, wr16)
            return 0
        lax.fori_loop(0, CHUNK // 16, selw, 0)

        @plsc.parallel_loop(0, CHUNK, unroll=4)
        def _edge(e):
            a = wA[b][pl.ds(e, 16)][0]
            bb = wB[b][pl.ds(e, 16)][0]
            for j in range(D // 16):
                sl = pl.ds(j * 16, 16)
                msg[b][e, sl] = a * gA[b][e, sl] + bb * gB[b][e, sl]

    def issue_scatter(b):
        _copy_idx(srcv[b], scx[b])
        pltpu.async_copy(msg[b], acc.at[scx[b]], sem_s[b], add=True)

    def wait_scatter(b):
        pltpu.make_async_copy(msg[b], acc.at[scx[b]], sem_s[b]).wait()

    loads(0, 0)
    loads(1, 1)
    wait_loads(0)
    issue_gathers(0)

    @pl.loop(0, NCHUNK, step=2)
    def _pipeline(k):
        for b in (0, 1):
            o = 1 - b
            kk = k + b

            @pl.when(kk + 1 < NCHUNK)
            def _():
                wait_loads(o)
                issue_gathers(o)

            @pl.when(kk >= 2)
            def _():
                wait_scatter(b)

            wait_gathers(b)
            compute(b)
            issue_scatter(b)

            @pl.when(kk + 2 < NCHUNK)
            def _():
                loads(kk + 2, b)

    wait_scatter(0)
    wait_scatter(1)
    plsc.subcore_barrier()

    @pl.when(s < NT - 1)
    def _():
        pltpu.sync_copy(acc.at[pl.ds(r0, ROWS_PT)],
                        out.at[pl.ds(c * NP + r0, ROWS_PT)])

    @pl.when(s == NT - 1)
    def _():
        pltpu.sync_copy(acc.at[pl.ds(LAST_R0, LAST_ROWS)],
                        out.at[pl.ds(c * NP + LAST_R0, LAST_ROWS)])


def _tc_body(xr, xi, s1a, s1b, u2, W, bias, outr, outi):
    W02 = W[0] - W[2]
    W1 = W[1]
    W22 = 2.0 * W[2]
    b = bias[0, :]

    rr = s1a[0]
    ri = s1a[1]
    ii = s1b[0]
    ir = s1b[1]

    dot = functools.partial(jnp.dot, preferred_element_type=jnp.float32)
    outr[...] = dot(xr[...], W02) + dot(ii - rr, W1) + dot(u2[0], W22) + b
    outi[...] = dot(xi[...], W02) - dot(ir + ri, W1) + dot(u2[1], W22) + b


def _sc_scratch_base():
    return [
        pltpu.VMEM_SHARED((NP, D), jnp.float32),
        pltpu.VMEM((CHUNK,), jnp.int32), pltpu.VMEM((CHUNK,), jnp.int32),
        pltpu.VMEM((CHUNK,), jnp.int32), pltpu.VMEM((CHUNK,), jnp.int32),
        pltpu.VMEM((CHUNK,), jnp.float32), pltpu.VMEM((CHUNK,), jnp.float32),
        pltpu.VMEM((CHUNK,), jnp.float32), pltpu.VMEM((CHUNK,), jnp.float32),
    ]


def _s1_scratch():
    return _sc_scratch_base() + [
        pltpu.VMEM((CHUNK + 16,), jnp.float32),
        pltpu.VMEM((CHUNK + 16,), jnp.float32),
        pltpu.VMEM((CHUNK, D), jnp.float32), pltpu.VMEM((CHUNK, D), jnp.float32),
        pltpu.VMEM((CHUNK, D), jnp.float32), pltpu.VMEM((CHUNK, D), jnp.float32),
        pltpu.VMEM((CHUNK,), jnp.int32), pltpu.VMEM((CHUNK,), jnp.int32),
    ] + [pltpu.SemaphoreType.DMA for _ in range(6)]


def _s2_scratch():
    return _sc_scratch_base() + [
        pltpu.VMEM((CHUNK + 16,), jnp.float32),
        pltpu.VMEM((CHUNK + 16,), jnp.float32),
        pltpu.VMEM((CHUNK + 16,), jnp.float32),
        pltpu.VMEM((CHUNK + 16,), jnp.float32),
        pltpu.VMEM((CHUNK, D), jnp.float32), pltpu.VMEM((CHUNK, D), jnp.float32),
        pltpu.VMEM((CHUNK, D), jnp.float32), pltpu.VMEM((CHUNK, D), jnp.float32),
        pltpu.VMEM((CHUNK, D), jnp.float32), pltpu.VMEM((CHUNK, D), jnp.float32),
        pltpu.VMEM((CHUNK,), jnp.int32), pltpu.VMEM((CHUNK,), jnp.int32),
    ] + [pltpu.SemaphoreType.DMA for _ in range(6)]


@jax.jit
def kernel(x_real, x_imag, edge_index, norm_real, norm_imag, W, bias):
    src = edge_index[0].astype(jnp.int32)
    dst = edge_index[1].astype(jnp.int32)
    wr = norm_real.astype(jnp.float32)
    wi = norm_imag.astype(jnp.float32)

    npad = E_PAD - E
    spread = (jnp.arange(npad, dtype=jnp.int32) * 61) % N
    dst_p = jnp.concatenate([dst, spread])
    src_p = jnp.concatenate([src, spread])
    zpad = jnp.zeros((npad,), jnp.float32)
    wr_p = jnp.concatenate([wr, zpad])
    wi_p = jnp.concatenate([wi, zpad])

    z128 = jnp.zeros((NP, D), jnp.float32)

    s1a = pl.kernel(
        functools.partial(_s1_body, False),
        out_type=jax.ShapeDtypeStruct((2 * NP, D), jnp.float32),
        mesh=_MESH, scratch_types=_s1_scratch(),
    )(x_real, dst_p, src_p, wr_p, wi_p, z128)
    s1b = pl.kernel(
        functools.partial(_s1_body, True),
        out_type=jax.ShapeDtypeStruct((2 * NP, D), jnp.float32),
        mesh=_MESH, scratch_types=_s1_scratch(),
    )(x_imag, dst_p, src_p, wr_p, wi_p, z128)

    u2 = pl.kernel(
        _s2_body,
        out_type=jax.ShapeDtypeStruct((2 * NP, D), jnp.float32),
        mesh=_MESH, scratch_types=_s2_scratch(),
    )(s1a, s1b, dst_p, src_p, wr_p, wi_p, z128)

    RB = 1000
    grid = N // RB
    out_real, out_imag = pl.pallas_call(
        _tc_body,
        grid=(grid,),
        in_specs=[
            pl.BlockSpec((RB, D), lambda i: (i, 0)),
            pl.BlockSpec((RB, D), lambda i: (i, 0)),
            pl.BlockSpec((2, RB, D), lambda i: (0, i, 0)),
            pl.BlockSpec((2, RB, D), lambda i: (0, i, 0)),
            pl.BlockSpec((2, RB, D), lambda i: (0, i, 0)),
            pl.BlockSpec((3, D, D), lambda i: (0, 0, 0)),
            pl.BlockSpec((1, D), lambda i: (0, 0)),
        ],
        out_specs=[
            pl.BlockSpec((RB, D), lambda i: (i, 0)),
            pl.BlockSpec((RB, D), lambda i: (i, 0)),
        ],
        out_shape=[
            jax.ShapeDtypeStruct((N, D), jnp.float32),
            jax.ShapeDtypeStruct((N, D), jnp.float32),
        ],
    )(x_real, x_imag,
      s1a.reshape(2, NP, D), s1b.reshape(2, NP, D),
      u2.reshape(2, NP, D), W, bias.reshape(1, D))

    return out_real, out_imag
